# baseline (speedup 1.0000x reference)
import functools

import jax
import jax.numpy as jnp
from jax import lax
from jax.experimental import pallas as pl
from jax.experimental.pallas import tpu as pltpu
from jax.experimental.pallas import tpu_sc as plsc

MAXLEN = 200
EMBED = 32
VOCAB = 1000000
BATCH = 4096
ROWS = BATCH * MAXLEN

NC, NS = 2, 16
NW = NC * NS

K1_TOK = 256
NBLK = VOCAB // K1_TOK
TAIL = VOCAB - NBLK * K1_TOK
K1_EXTRA = NBLK - (NBLK // NW) * NW
K1_CNT = NBLK // NW
NBUF = 4


PITCH = 257


def _k1_body(tt_ref, tail_ref, rm_ref, bin_v, bout_v, *sems):
    wid = lax.axis_index("s") * NC + lax.axis_index("c")
    cnt = jnp.where(wid < K1_EXTRA, K1_CNT + 1, K1_CNT)
    start = wid * K1_CNT + jnp.minimum(wid, K1_EXTRA)
    iota = lax.iota(jnp.int32, 16)
    isems, ssems = sems[:NBUF], sems[NBUF:]
    orows = K1_TOK * EMBED // 128

    def do_block(i, b):
        @pl.when(i < cnt)
        def _():
            j = start + i
            src = tt_ref.at[:, pl.ds(j * K1_TOK, K1_TOK)]
            dst = rm_ref.at[pl.ds(j * orows, orows)]
            bpad = bin_v.at[b, :, pl.ds(0, K1_TOK)]

            @pl.when(i >= NBUF)
            def _():
                pltpu.make_async_copy(bout_v.at[b], dst, ssems[b]).wait()
                pltpu.make_async_copy(src, bpad, isems[b]).wait()

            @pl.when(i < NBUF)
            def _():
                pltpu.sync_copy(src, bpad)

            def tstep(q, acc):
                vecs = []
                for tt in range(4):
                    bc = jnp.broadcast_to(q * 4 + tt, (16,))
                    for h in range(2):
                        vecs.append(
                            plsc.load_gather(bin_v.at[b], [iota + (h * 16), bc])
                        )
                for n, vec in enumerate(vecs):
                    bout_v[b, q, pl.ds(n * 16, 16)] = vec
                return acc

            lax.fori_loop(0, K1_TOK // 4, tstep, 0)

            @pl.when(i + NBUF < cnt)
            def _():
                nsrc = tt_ref.at[:, pl.ds((start + i + NBUF) * K1_TOK, K1_TOK)]
                pltpu.async_copy(nsrc, bpad, isems[b])

            pltpu.async_copy(bout_v.at[b], dst, ssems[b])

    def grp(k, carry):
        for b in range(NBUF):
            do_block(NBUF * k + b, b)
        return carry

    lax.fori_loop(0, (K1_CNT + NBUF) // NBUF, grp, 0)
    for b in range(NBUF):
        pltpu.make_async_copy(
            bout_v.at[b], rm_ref.at[pl.ds(0, orows)], ssems[b]
        ).wait()

    @pl.when(wid == NW - 1)
    def _():
        pltpu.sync_copy(tail_ref, rm_ref.at[pl.ds(NBLK * orows, TAIL * EMBED // 128)])


K2_UNITS = ROWS // 1024
K2_PER_W = K2_UNITS // NW


def _k2_body(valt_ref, rm_ref, pat_ref, mid_ref, idx_v, buf_v, pat_v, sem, osem0, osem1):
    wid = lax.axis_index("s") * NC + lax.axis_index("c")
    pltpu.sync_copy(pat_ref, pat_v)
    base_u = wid * K2_PER_W

    def do_unit(i, b, osem):
        u = base_u + i
        s = u // 4
        prow = s // 4
        pcol = (s % 4) * EMBED
        pltpu.sync_copy(valt_ref.at[pl.ds(u * 8, 8)], idx_v.at[b])
        copies = []
        for j in range(8):
            copies.append(
                pltpu.async_copy(
                    rm_ref.at[idx_v.at[b, j]],
                    buf_v.at[b, pl.ds(j * 128, 128)],
                    sem,
                )
            )
        for cp in copies:
            cp.wait()
        pv0 = pat_v[prow, pl.ds(pcol, 16)]
        pv1 = pat_v[prow, pl.ds(pcol + 16, 16)]

        def add_step(q, acc):
            for rr in range(4):
                r = q * 4 + rr
                plsc.addupdate(buf_v.at[b, r, pl.ds(0, 16)], pv0)
                plsc.addupdate(buf_v.at[b, r, pl.ds(16, 16)], pv1)
            return acc

        lax.fori_loop(0, 256, add_step, 0)
        return pltpu.async_copy(buf_v.at[b], mid_ref.at[pl.ds(u * 1024, 1024)], osem)

    sems = [osem0, osem1]
    handles = [None, None]
    for i in range(K2_PER_W):
        b = i % 2
        if handles[b] is not None:
            handles[b].wait()
        handles[b] = do_unit(i, b, sems[b])
    handles[0].wait()
    handles[1].wait()


K3_JJ = 2
K3_UNITS = MAXLEN * (BATCH // (128 * K3_JJ))
K3_PER_W = K3_UNITS // NW
K3_PITCH = 129


def _k3_body(mid_ref, out_ref, bin_v, bout_v, *sems):
    wid = lax.axis_index("s") * NC + lax.axis_index("c")
    base_u = wid * K3_PER_W
    iota = lax.iota(jnp.int32, 16)
    isems, ssems = sems[:NBUF], sems[NBUF:]
    nrows = 32 * K3_JJ

    def unit_src(u):
        return mid_ref.at[pl.ds(u * nrows, nrows)]

    def do_unit(i, b):
        u = base_u + i
        s = u // (32 // K3_JJ)
        jcol = (u % (32 // K3_JJ)) * (128 * K3_JJ)

        @pl.when(i >= NBUF)
        def _():
            for jj in range(K3_JJ):
                pltpu.make_async_copy(
                    bout_v.at[b, jj, :, pl.ds(0, 128)],
                    out_ref.at[s, :, pl.ds(jcol + jj * 128, 128)],
                    ssems[b],
                ).wait()
            pltpu.make_async_copy(unit_src(u), bin_v.at[b], isems[b]).wait()

        @pl.when(i < NBUF)
        def _():
            pltpu.sync_copy(unit_src(u), bin_v.at[b])

        for jj in range(K3_JJ):
            def tstep(q, acc, jj=jj):
                vecs = []
                for n in range(8):
                    vecs.append(bin_v[b, jj * 32 + q, pl.ds(n * 16, 16)])
                for tt in range(4):
                    bc = jnp.broadcast_to(q * 4 + tt, (16,))
                    for h in range(2):
                        plsc.store_scatter(
                            bout_v.at[b, jj], [iota + (h * 16), bc],
                            vecs[tt * 2 + h],
                        )
                return acc

            lax.fori_loop(0, 32, tstep, 0)

        @pl.when(i + NBUF < K3_PER_W)
        def _():
            pltpu.async_copy(unit_src(u + NBUF), bin_v.at[b], isems[b])

        for jj in range(K3_JJ):
            pltpu.async_copy(
                bout_v.at[b, jj, :, pl.ds(0, 128)],
                out_ref.at[s, :, pl.ds(jcol + jj * 128, 128)],
                ssems[b],
            )

    def grp(k, carry):
        for b in range(NBUF):
            do_unit(NBUF * k + b, b)
        return carry

    lax.fori_loop(0, K3_PER_W // NBUF, grp, 0)
    for b in range(NBUF):
        for jj in range(K3_JJ):
            pltpu.make_async_copy(
                bout_v.at[b, jj, :, pl.ds(0, 128)],
                out_ref.at[0, :, pl.ds(jj * 128, 128)],
                ssems[b],
            ).wait()


_MESH = plsc.VectorSubcoreMesh(core_axis_name="c", subcore_axis_name="s")


@jax.jit
def _run(val, token_table, pos_table):
    tt = token_table.T
    tail = token_table[NBLK * K1_TOK:].reshape(TAIL * EMBED // 128, 128)
    valt = val.T.astype(jnp.int32).reshape(ROWS // 128, 128)
    patq = pos_table.reshape(MAXLEN * EMBED // 128, 128)

    k1 = functools.partial(
        pl.kernel,
        mesh=_MESH,
        out_type=jax.ShapeDtypeStruct((VOCAB * EMBED // 128, 128), jnp.float32),
        scratch_types=[
            pltpu.VMEM((NBUF, EMBED, PITCH), jnp.float32),
            pltpu.VMEM((NBUF, K1_TOK // 4, 128), jnp.float32),
        ] + [pltpu.SemaphoreType.DMA] * (2 * NBUF),
        compiler_params=pltpu.CompilerParams(needs_layout_passes=False),
    )(_k1_body)
    rm4 = k1(tt, tail)

    k2 = functools.partial(
        pl.kernel,
        mesh=_MESH,
        out_type=jax.ShapeDtypeStruct((ROWS, EMBED), jnp.float32),
        scratch_types=[
            pltpu.VMEM((2, 8, 128), jnp.int32),
            pltpu.VMEM((2, 1024, EMBED), jnp.float32),
            pltpu.VMEM((MAXLEN * EMBED // 128, 128), jnp.float32),
            pltpu.SemaphoreType.DMA,
            pltpu.SemaphoreType.DMA,
            pltpu.SemaphoreType.DMA,
        ],
        compiler_params=pltpu.CompilerParams(use_tc_tiling_on_sc=False),
    )(_k2_body)
    mid = k2(valt, rm4.reshape(VOCAB, EMBED), patq)

    k3 = functools.partial(
        pl.kernel,
        mesh=_MESH,
        out_type=jax.ShapeDtypeStruct((MAXLEN, EMBED, BATCH), jnp.float32),
        scratch_types=[
            pltpu.VMEM((NBUF, 32 * K3_JJ, 128), jnp.float32),
            pltpu.VMEM((NBUF, K3_JJ, EMBED, K3_PITCH), jnp.float32),
        ] + [pltpu.SemaphoreType.DMA] * (2 * NBUF),
        compiler_params=pltpu.CompilerParams(needs_layout_passes=False),
    )(_k3_body)
    outt = k3(mid.reshape(ROWS * EMBED // 128, 128))
    return jnp.transpose(outt, (2, 0, 1))


def kernel(val, token_table, pos_table):
    return _run(val, token_table, pos_table)


# speedup vs baseline: 1.0016x; 1.0016x over previous
import functools

import jax
import jax.numpy as jnp
from jax import lax
from jax.experimental import pallas as pl
from jax.experimental.pallas import tpu as pltpu
from jax.experimental.pallas import tpu_sc as plsc

MAXLEN = 200
EMBED = 32
VOCAB = 1000000
BATCH = 4096
ROWS = BATCH * MAXLEN

NC, NS = 2, 16
NW = NC * NS

K1_TOK = 768
NBLK = VOCAB // K1_TOK
TAIL = VOCAB - NBLK * K1_TOK
K1_EXTRA = NBLK - (NBLK // NW) * NW
K1_CNT = NBLK // NW
K1_BUF = 2
NBUF = 4


PITCH = K1_TOK + 1


def _k1_body(tt_ref, tail_ref, rm_ref, bin_v, bout_v, *sems):
    wid = lax.axis_index("s") * NC + lax.axis_index("c")
    cnt = jnp.where(wid < K1_EXTRA, K1_CNT + 1, K1_CNT)
    start = wid * K1_CNT + jnp.minimum(wid, K1_EXTRA)
    iota = lax.iota(jnp.int32, 16)
    isems, ssems = sems[:K1_BUF], sems[K1_BUF:]
    orows = K1_TOK * EMBED // 128

    def do_block(i, b):
        @pl.when(i < cnt)
        def _():
            j = start + i
            src = tt_ref.at[:, pl.ds(j * K1_TOK, K1_TOK)]
            dst = rm_ref.at[pl.ds(j * orows, orows)]
            bpad = bin_v.at[b, :, pl.ds(0, K1_TOK)]

            @pl.when(i >= K1_BUF)
            def _():
                pltpu.make_async_copy(bout_v.at[b], dst, ssems[b]).wait()
                pltpu.make_async_copy(src, bpad, isems[b]).wait()

            @pl.when(i < K1_BUF)
            def _():
                pltpu.sync_copy(src, bpad)

            def tstep(q, acc):
                vecs = []
                for tt in range(4):
                    bc = jnp.broadcast_to(q * 4 + tt, (16,))
                    for h in range(2):
                        vecs.append(
                            plsc.load_gather(bin_v.at[b], [iota + (h * 16), bc])
                        )
                for n, vec in enumerate(vecs):
                    bout_v[b, q, pl.ds(n * 16, 16)] = vec
                return acc

            lax.fori_loop(0, K1_TOK // 4, tstep, 0)

            @pl.when(i + K1_BUF < cnt)
            def _():
                nsrc = tt_ref.at[:, pl.ds((start + i + K1_BUF) * K1_TOK, K1_TOK)]
                pltpu.async_copy(nsrc, bpad, isems[b])

            pltpu.async_copy(bout_v.at[b], dst, ssems[b])

    def grp(k, carry):
        for b in range(K1_BUF):
            do_block(K1_BUF * k + b, b)
        return carry

    lax.fori_loop(0, (K1_CNT + K1_BUF) // K1_BUF, grp, 0)
    for b in range(K1_BUF):
        pltpu.make_async_copy(
            bout_v.at[b], rm_ref.at[pl.ds(0, orows)], ssems[b]
        ).wait()

    @pl.when(wid == NW - 1)
    def _():
        pltpu.sync_copy(tail_ref, rm_ref.at[pl.ds(NBLK * orows, TAIL * EMBED // 128)])


K2_UNITS = ROWS // 1024
K2_PER_W = K2_UNITS // NW


def _k2_body(valt_ref, rm_ref, pat_ref, mid_ref, idx_v, buf_v, pat_v, sem, osem0, osem1):
    wid = lax.axis_index("s") * NC + lax.axis_index("c")
    pltpu.sync_copy(pat_ref, pat_v)
    base_u = wid * K2_PER_W

    def do_unit(i, b, osem):
        u = base_u + i
        s = u // 4
        prow = s // 4
        pcol = (s % 4) * EMBED
        pltpu.sync_copy(valt_ref.at[pl.ds(u * 8, 8)], idx_v.at[b])
        copies = []
        for j in range(8):
            copies.append(
                pltpu.async_copy(
                    rm_ref.at[idx_v.at[b, j]],
                    buf_v.at[b, pl.ds(j * 128, 128)],
                    sem,
                )
            )
        for cp in copies:
            cp.wait()
        pv0 = pat_v[prow, pl.ds(pcol, 16)]
        pv1 = pat_v[prow, pl.ds(pcol + 16, 16)]

        def add_step(q, acc):
            for rr in range(4):
                r = q * 4 + rr
                plsc.addupdate(buf_v.at[b, r, pl.ds(0, 16)], pv0)
                plsc.addupdate(buf_v.at[b, r, pl.ds(16, 16)], pv1)
            return acc

        lax.fori_loop(0, 256, add_step, 0)
        return pltpu.async_copy(buf_v.at[b], mid_ref.at[pl.ds(u * 1024, 1024)], osem)

    sems = [osem0, osem1]
    handles = [None, None]
    for i in range(K2_PER_W):
        b = i % 2
        if handles[b] is not None:
            handles[b].wait()
        handles[b] = do_unit(i, b, sems[b])
    handles[0].wait()
    handles[1].wait()


K3_JJ = 2
K3_UNITS = MAXLEN * (BATCH // (128 * K3_JJ))
K3_PER_W = K3_UNITS // NW
K3_PITCH = 128 * K3_JJ + 1


def _k3_body(mid_ref, out_ref, bin_v, bout_v, *sems):
    wid = lax.axis_index("s") * NC + lax.axis_index("c")
    base_u = wid * K3_PER_W
    iota = lax.iota(jnp.int32, 16)
    isems, ssems = sems[:NBUF], sems[NBUF:]
    nrows = 32 * K3_JJ

    def unit_src(u):
        return mid_ref.at[pl.ds(u * nrows, nrows)]

    def do_unit(i, b):
        u = base_u + i
        s = u // (32 // K3_JJ)
        jcol = (u % (32 // K3_JJ)) * (128 * K3_JJ)

        @pl.when(i >= NBUF)
        def _():
            pltpu.make_async_copy(
                bout_v.at[b, :, pl.ds(0, 128 * K3_JJ)],
                out_ref.at[s, :, pl.ds(jcol, 128 * K3_JJ)],
                ssems[b],
            ).wait()
            pltpu.make_async_copy(unit_src(u), bin_v.at[b], isems[b]).wait()

        @pl.when(i < NBUF)
        def _():
            pltpu.sync_copy(unit_src(u), bin_v.at[b])

        for jj in range(K3_JJ):
            def tstep(q, acc, jj=jj):
                vecs = []
                for n in range(8):
                    vecs.append(bin_v[b, jj * 32 + q, pl.ds(n * 16, 16)])
                for tt in range(4):
                    bc = jnp.broadcast_to(jj * 128 + q * 4 + tt, (16,))
                    for h in range(2):
                        plsc.store_scatter(
                            bout_v.at[b], [iota + (h * 16), bc],
                            vecs[tt * 2 + h],
                        )
                return acc

            lax.fori_loop(0, 32, tstep, 0)

        @pl.when(i + NBUF < K3_PER_W)
        def _():
            pltpu.async_copy(unit_src(u + NBUF), bin_v.at[b], isems[b])

        pltpu.async_copy(
            bout_v.at[b, :, pl.ds(0, 128 * K3_JJ)],
            out_ref.at[s, :, pl.ds(jcol, 128 * K3_JJ)],
            ssems[b],
        )

    def grp(k, carry):
        for b in range(NBUF):
            do_unit(NBUF * k + b, b)
        return carry

    lax.fori_loop(0, K3_PER_W // NBUF, grp, 0)
    for b in range(NBUF):
        pltpu.make_async_copy(
            bout_v.at[b, :, pl.ds(0, 128 * K3_JJ)],
            out_ref.at[0, :, pl.ds(0, 128 * K3_JJ)],
            ssems[b],
        ).wait()


_MESH = plsc.VectorSubcoreMesh(core_axis_name="c", subcore_axis_name="s")


@jax.jit
def _run(val, token_table, pos_table):
    tt = token_table.T
    tail = token_table[NBLK * K1_TOK:].reshape(TAIL * EMBED // 128, 128)
    valt = val.T.astype(jnp.int32).reshape(ROWS // 128, 128)
    patq = pos_table.reshape(MAXLEN * EMBED // 128, 128)

    k1 = functools.partial(
        pl.kernel,
        mesh=_MESH,
        out_type=jax.ShapeDtypeStruct((VOCAB * EMBED // 128, 128), jnp.float32),
        scratch_types=[
            pltpu.VMEM((K1_BUF, EMBED, PITCH), jnp.float32),
            pltpu.VMEM((K1_BUF, K1_TOK // 4, 128), jnp.float32),
        ] + [pltpu.SemaphoreType.DMA] * (2 * K1_BUF),
        compiler_params=pltpu.CompilerParams(needs_layout_passes=False),
    )(_k1_body)
    rm4 = k1(tt, tail)

    k2 = functools.partial(
        pl.kernel,
        mesh=_MESH,
        out_type=jax.ShapeDtypeStruct((ROWS, EMBED), jnp.float32),
        scratch_types=[
            pltpu.VMEM((2, 8, 128), jnp.int32),
            pltpu.VMEM((2, 1024, EMBED), jnp.float32),
            pltpu.VMEM((MAXLEN * EMBED // 128, 128), jnp.float32),
            pltpu.SemaphoreType.DMA,
            pltpu.SemaphoreType.DMA,
            pltpu.SemaphoreType.DMA,
        ],
        compiler_params=pltpu.CompilerParams(use_tc_tiling_on_sc=False),
    )(_k2_body)
    mid = k2(valt, rm4.reshape(VOCAB, EMBED), patq)

    k3 = functools.partial(
        pl.kernel,
        mesh=_MESH,
        out_type=jax.ShapeDtypeStruct((MAXLEN, EMBED, BATCH), jnp.float32),
        scratch_types=[
            pltpu.VMEM((NBUF, 32 * K3_JJ, 128), jnp.float32),
            pltpu.VMEM((NBUF, EMBED, K3_PITCH), jnp.float32),
        ] + [pltpu.SemaphoreType.DMA] * (2 * NBUF),
        compiler_params=pltpu.CompilerParams(needs_layout_passes=False),
    )(_k3_body)
    outt = k3(mid.reshape(ROWS * EMBED // 128, 128))
    return jnp.transpose(outt, (2, 0, 1))


def kernel(val, token_table, pos_table):
    return _run(val, token_table, pos_table)
